# Initial kernel scaffold; baseline (speedup 1.0000x reference)
import jax
import jax.numpy as jnp
from jax.experimental import pallas as pl


def kernel(input_ids, pos_table):
    raise NotImplementedError("write your pallas kernel here")


# speedup vs baseline: 3.4426x; 3.4426x over previous
import jax
import jax.numpy as jnp
from jax.experimental import pallas as pl

BLK_S = 512


def _bcast_body(table_ref, out_ref):
    out_ref[...] = table_ref[...][None]


def kernel(input_ids, pos_table):
    batch, seq_len = input_ids.shape
    d_model = pos_table.shape[1]
    grid = (seq_len // BLK_S, batch)
    out = pl.pallas_call(
        _bcast_body,
        grid=grid,
        in_specs=[pl.BlockSpec((BLK_S, d_model), lambda i, j: (i, 0))],
        out_specs=pl.BlockSpec((1, BLK_S, d_model), lambda i, j: (j, i, 0)),
        out_shape=jax.ShapeDtypeStruct((batch, seq_len, d_model), pos_table.dtype),
    )(pos_table)
    return out


# speedup vs baseline: 4.9964x; 1.4513x over previous
import jax
import jax.numpy as jnp
from jax.experimental import pallas as pl

BLK_S = 512


def _bcast_body(table_ref, out_ref):
    out_ref[...] = jnp.broadcast_to(table_ref[...][None], out_ref.shape)


def kernel(input_ids, pos_table):
    batch, seq_len = input_ids.shape
    d_model = pos_table.shape[1]
    grid = (seq_len // BLK_S,)
    out = pl.pallas_call(
        _bcast_body,
        grid=grid,
        in_specs=[pl.BlockSpec((BLK_S, d_model), lambda i: (i, 0))],
        out_specs=pl.BlockSpec((batch, BLK_S, d_model), lambda i: (0, i, 0)),
        out_shape=jax.ShapeDtypeStruct((batch, seq_len, d_model), pos_table.dtype),
    )(pos_table)
    return out


# speedup vs baseline: 5.1599x; 1.0327x over previous
import jax
import jax.numpy as jnp
from jax.experimental import pallas as pl

BLK_S = 1024


def _bcast_body(table_ref, out_ref):
    out_ref[...] = jnp.broadcast_to(table_ref[...][None], out_ref.shape)


def kernel(input_ids, pos_table):
    batch, seq_len = input_ids.shape
    d_model = pos_table.shape[1]
    grid = (seq_len // BLK_S,)
    out = pl.pallas_call(
        _bcast_body,
        grid=grid,
        in_specs=[pl.BlockSpec((BLK_S, d_model), lambda i: (i, 0))],
        out_specs=pl.BlockSpec((batch, BLK_S, d_model), lambda i: (0, i, 0)),
        out_shape=jax.ShapeDtypeStruct((batch, seq_len, d_model), pos_table.dtype),
    )(pos_table)
    return out
